# baseline (speedup 1.0000x reference)
import functools

import jax
import jax.numpy as jnp
from jax import lax
from jax.experimental import pallas as pl
from jax.experimental.pallas import tpu as pltpu
from jax.experimental.pallas import tpu_sc as plsc

_B = 8
_H = 384
_W = 384
_P = 5000
_PPAD = 5120
_NPAIR = _B * _PPAD
_CHUNK = 1280
_NROW = _CHUNK // 128
_NW = 32
_VALID_TAIL = _P - 3 * _CHUNK
_OFF_MASK = (1 << 18) - 1

_MARGIN = 0.25
_LN2 = 0.6931471805599453
_SQRT2 = 1.4142135623730951
_INV_P = 1.0 / _P


def _vlog(x):
    bits = lax.bitcast_convert_type(x, jnp.int32)
    e = lax.shift_right_arithmetic(bits, 23) - 127
    m_bits = lax.bitwise_or(lax.bitwise_and(bits, 0x007FFFFF), 0x3F800000)
    m = lax.bitcast_convert_type(m_bits, jnp.float32)
    big = m >= jnp.float32(_SQRT2)
    m = jnp.where(big, m * jnp.float32(0.5), m)
    e = e + jnp.where(big, 1, 0)
    ef = e.astype(jnp.float32)
    t = (m - 1.0) / (m + 1.0)
    t2 = t * t
    p = jnp.float32(1.0 / 9.0)
    p = p * t2 + jnp.float32(1.0 / 7.0)
    p = p * t2 + jnp.float32(1.0 / 5.0)
    p = p * t2 + jnp.float32(1.0 / 3.0)
    p = p * t2 + jnp.float32(1.0)
    return ef * jnp.float32(_LN2) + (t + t) * p


def _vlog1p(u):
    t = u / (u + 2.0)
    t2 = t * t
    p = jnp.float32(1.0 / 11.0)
    p = p * t2 + jnp.float32(1.0 / 9.0)
    p = p * t2 + jnp.float32(1.0 / 7.0)
    p = p * t2 + jnp.float32(1.0 / 5.0)
    p = p * t2 + jnp.float32(1.0 / 3.0)
    p = p * t2 + jnp.float32(1.0)
    return (t + t) * p


def _tec_body(img, pa, pb, out, pav, pbv, iav, ibv, vav, vbv, resv, sem, gsem):
    cid = lax.axis_index("c")
    sid = lax.axis_index("s")
    wid = sid * 2 + cid
    base = wid * _CHUNK
    imgbase = (wid // 4) * (_H * _W)
    limit = jnp.where(wid % 4 == 3, _VALID_TAIL, _CHUNK)

    ca = pltpu.async_copy(pa.at[pl.ds(base, _CHUNK)], pav, sem)
    cb = pltpu.async_copy(pb.at[pl.ds(base, _CHUNK)], pbv, sem)
    ca.wait()
    cb.wait()

    for j in range(_NROW):
        for kk in range(8):
            off = j * 128 + kk * 16
            iav[j, pl.ds(kk * 16, 16)] = (
                imgbase + (pav[pl.ds(off, 16)] & _OFF_MASK))
            ibv[j, pl.ds(kk * 16, 16)] = (
                imgbase + (pbv[pl.ds(off, 16)] & _OFF_MASK))
        pltpu.async_copy(img.at[iav.at[j]], vav.at[j], gsem.at[j])
        pltpu.async_copy(img.at[ibv.at[j]], vbv.at[j], gsem.at[j])

    lane = lax.iota(jnp.int32, 16)

    def comp_body(j, acc):
        pltpu.make_async_copy(img.at[iav.at[j]], vav.at[j], gsem.at[j]).wait()
        pltpu.make_async_copy(img.at[ibv.at[j]], vbv.at[j], gsem.at[j]).wait()
        for kk in range(8):
            off = kk * 16
            a = vav[j, pl.ds(kk * 16, 16)]
            b = vbv[j, pl.ds(kk * 16, 16)]
            o = lax.shift_right_logical(pav[pl.ds(j * 128 + kk * 16, 16)], 18)
            diff = _vlog(a / b)
            r = (o - 1).astype(jnp.float32)
            t = jnp.float32(_MARGIN) - r * diff
            u = jnp.exp(-jnp.abs(t))
            sp = jnp.maximum(t, 0.0) + _vlog1p(u)
            eq = jnp.maximum(diff * diff - jnp.float32(_MARGIN), 0.0)
            per = jnp.where(o == 1, eq, sp)
            per = jnp.where(j * 128 + off + lane < limit, per, 0.0)
            acc = acc + per
        return acc

    acc = lax.fori_loop(0, _NROW, comp_body, jnp.zeros((16,), jnp.float32))

    resv[...] = acc * jnp.float32(_INV_P)
    pltpu.sync_copy(resv, out.at[wid])


@functools.partial(
    pl.kernel,
    mesh=plsc.VectorSubcoreMesh(core_axis_name="c", subcore_axis_name="s"),
    out_type=jax.ShapeDtypeStruct((_NW, 16), jnp.float32),
    scratch_types=[
        pltpu.VMEM((_CHUNK,), jnp.int32),
        pltpu.VMEM((_CHUNK,), jnp.int32),
        pltpu.VMEM((_NROW, 128), jnp.int32),
        pltpu.VMEM((_NROW, 128), jnp.int32),
        pltpu.VMEM((_NROW, 128), jnp.float32),
        pltpu.VMEM((_NROW, 128), jnp.float32),
        pltpu.VMEM((16,), jnp.float32),
        pltpu.SemaphoreType.DMA,
        pltpu.SemaphoreType.DMA((_NROW,)),
    ],
)
def _sc_loss(img, pa, pb, out, *scratch):
    _tec_body(img, pa, pb, out, *scratch)


def _pad_flat(a):
    return jnp.pad(a, ((0, 0), (0, _PPAD - _P))).reshape(-1)


def kernel(input, y_A, x_A, y_B, x_B, ordinal):
    img = input.reshape(-1)
    pa = _pad_flat(y_A * _W + x_A | (ordinal << 18))
    pb = _pad_flat(y_B * _W + x_B)
    partials = _sc_loss(img, pa, pb)
    return partials


# speedup vs baseline: 1.0121x; 1.0121x over previous
import functools

import jax
import jax.numpy as jnp
from jax import lax
from jax.experimental import pallas as pl
from jax.experimental.pallas import tpu as pltpu
from jax.experimental.pallas import tpu_sc as plsc

_B = 8
_H = 384
_W = 384
_P = 5000
_PPAD = 5120
_NPAIR = _B * _PPAD
_CHUNK = 1280
_NROW = _CHUNK // 128
_NW = 32
_VALID_TAIL = _P - 3 * _CHUNK
_OFF_MASK = (1 << 18) - 1

_MARGIN = 0.25
_LN2 = 0.6931471805599453
_SQRT2 = 1.4142135623730951
_INV_P = 1.0 / _P


def _vlog(x):
    bits = lax.bitcast_convert_type(x, jnp.int32)
    e = lax.shift_right_arithmetic(bits, 23) - 127
    m_bits = lax.bitwise_or(lax.bitwise_and(bits, 0x007FFFFF), 0x3F800000)
    m = lax.bitcast_convert_type(m_bits, jnp.float32)
    big = m >= jnp.float32(_SQRT2)
    m = jnp.where(big, m * jnp.float32(0.5), m)
    e = e + jnp.where(big, 1, 0)
    ef = e.astype(jnp.float32)
    t = (m - 1.0) / (m + 1.0)
    t2 = t * t
    p = jnp.float32(1.0 / 9.0)
    p = p * t2 + jnp.float32(1.0 / 7.0)
    p = p * t2 + jnp.float32(1.0 / 5.0)
    p = p * t2 + jnp.float32(1.0 / 3.0)
    p = p * t2 + jnp.float32(1.0)
    return ef * jnp.float32(_LN2) + (t + t) * p


def _vlog1p(u):
    t = u / (u + 2.0)
    t2 = t * t
    p = jnp.float32(1.0 / 11.0)
    p = p * t2 + jnp.float32(1.0 / 9.0)
    p = p * t2 + jnp.float32(1.0 / 7.0)
    p = p * t2 + jnp.float32(1.0 / 5.0)
    p = p * t2 + jnp.float32(1.0 / 3.0)
    p = p * t2 + jnp.float32(1.0)
    return (t + t) * p


def _tec_body(img, pa, pb, out, pav, pbv, iav, ibv, vav, vbv, resv, sem, gsem):
    cid = lax.axis_index("c")
    sid = lax.axis_index("s")
    wid = sid * 2 + cid
    base = wid * _CHUNK
    imgbase = (wid // 4) * (_H * _W)
    limit = jnp.where(wid % 4 == 3, _VALID_TAIL, _CHUNK)

    ca = pltpu.async_copy(pa.at[pl.ds(base, _CHUNK)], pav, sem)
    cb = pltpu.async_copy(pb.at[pl.ds(base, _CHUNK)], pbv, sem)
    ca.wait()
    cb.wait()

    for j in range(_NROW):
        for kk in range(8):
            off = j * 128 + kk * 16
            iav[pl.ds(off, 16)] = imgbase + (pav[pl.ds(off, 16)] & _OFF_MASK)
            ibv[pl.ds(off, 16)] = imgbase + (pbv[pl.ds(off, 16)] & _OFF_MASK)

    ga = pltpu.async_copy(img.at[iav], vav, sem)
    gb = pltpu.async_copy(img.at[ibv], vbv, sem)
    ga.wait()
    gb.wait()

    lane = lax.iota(jnp.int32, 16)

    def comp_body(j, acc):
        for kk in range(8):
            off = kk * 16
            a = vav[pl.ds(j * 128 + kk * 16, 16)]
            b = vbv[pl.ds(j * 128 + kk * 16, 16)]
            o = lax.shift_right_logical(pav[pl.ds(j * 128 + kk * 16, 16)], 18)
            diff = _vlog(a / b)
            r = (o - 1).astype(jnp.float32)
            t = jnp.float32(_MARGIN) - r * diff
            u = jnp.exp(-jnp.abs(t))
            sp = jnp.maximum(t, 0.0) + _vlog1p(u)
            eq = jnp.maximum(diff * diff - jnp.float32(_MARGIN), 0.0)
            per = jnp.where(o == 1, eq, sp)
            per = jnp.where(j * 128 + off + lane < limit, per, 0.0)
            acc = acc + per
        return acc

    acc = lax.fori_loop(0, _NROW, comp_body, jnp.zeros((16,), jnp.float32))

    resv[...] = acc * jnp.float32(_INV_P)
    pltpu.sync_copy(resv, out.at[wid])


@functools.partial(
    pl.kernel,
    mesh=plsc.VectorSubcoreMesh(core_axis_name="c", subcore_axis_name="s"),
    out_type=jax.ShapeDtypeStruct((_NW, 16), jnp.float32),
    scratch_types=[
        pltpu.VMEM((_CHUNK,), jnp.int32),
        pltpu.VMEM((_CHUNK,), jnp.int32),
        pltpu.VMEM((_CHUNK,), jnp.int32),
        pltpu.VMEM((_CHUNK,), jnp.int32),
        pltpu.VMEM((_CHUNK,), jnp.float32),
        pltpu.VMEM((_CHUNK,), jnp.float32),
        pltpu.VMEM((16,), jnp.float32),
        pltpu.SemaphoreType.DMA,
        pltpu.SemaphoreType.DMA((_NROW,)),
    ],
)
def _sc_loss(img, pa, pb, out, *scratch):
    _tec_body(img, pa, pb, out, *scratch)


def _pad_flat(a):
    return jnp.pad(a, ((0, 0), (0, _PPAD - _P))).reshape(-1)


def kernel(input, y_A, x_A, y_B, x_B, ordinal):
    img = input.reshape(-1)
    pa = _pad_flat(y_A * _W + x_A | (ordinal << 18))
    pb = _pad_flat(y_B * _W + x_B)
    partials = _sc_loss(img, pa, pb)
    return jnp.sum(partials)


# speedup vs baseline: 1.2291x; 1.2145x over previous
import functools

import jax
import jax.numpy as jnp
from jax import lax
from jax.experimental import pallas as pl
from jax.experimental.pallas import tpu as pltpu
from jax.experimental.pallas import tpu_sc as plsc

_B = 8
_H = 384
_W = 384
_P = 5000
_PPAD = 5120
_NPAIR = _B * _PPAD
_CHUNK = 1280
_NROW = _CHUNK // 128
_NW = 32
_VALID_TAIL = _P - 3 * _CHUNK
_OFF_MASK = (1 << 18) - 1

_MARGIN = 0.25
_LN2 = 0.6931471805599453
_SQRT2 = 1.4142135623730951
_INV_P = 1.0 / _P


def _vlog(x):
    bits = lax.bitcast_convert_type(x, jnp.int32)
    e = lax.shift_right_arithmetic(bits, 23) - 127
    m_bits = lax.bitwise_or(lax.bitwise_and(bits, 0x007FFFFF), 0x3F800000)
    m = lax.bitcast_convert_type(m_bits, jnp.float32)
    big = m >= jnp.float32(_SQRT2)
    m = jnp.where(big, m * jnp.float32(0.5), m)
    e = e + jnp.where(big, 1, 0)
    ef = e.astype(jnp.float32)
    t = (m - 1.0) / (m + 1.0)
    t2 = t * t
    p = jnp.float32(1.0 / 9.0)
    p = p * t2 + jnp.float32(1.0 / 7.0)
    p = p * t2 + jnp.float32(1.0 / 5.0)
    p = p * t2 + jnp.float32(1.0 / 3.0)
    p = p * t2 + jnp.float32(1.0)
    return ef * jnp.float32(_LN2) + (t + t) * p


def _vlog1p(u):
    t = u / (u + 2.0)
    t2 = t * t
    p = jnp.float32(1.0 / 11.0)
    p = p * t2 + jnp.float32(1.0 / 9.0)
    p = p * t2 + jnp.float32(1.0 / 7.0)
    p = p * t2 + jnp.float32(1.0 / 5.0)
    p = p * t2 + jnp.float32(1.0 / 3.0)
    p = p * t2 + jnp.float32(1.0)
    return (t + t) * p


def _tec_body(img, pa, pb, out, pav, pbv, iav, ibv, vav, vbv, resv, sem, gsem):
    cid = lax.axis_index("c")
    sid = lax.axis_index("s")
    wid = sid * 2 + cid
    base = wid * _CHUNK
    imgbase = (wid // 4) * (_H * _W)
    limit = jnp.where(wid % 4 == 3, _VALID_TAIL, _CHUNK)

    ca = pltpu.async_copy(pa.at[pl.ds(base, _CHUNK)], pav, sem)
    cb = pltpu.async_copy(pb.at[pl.ds(base, _CHUNK)], pbv, sem)
    ca.wait()
    cb.wait()

    for j in range(_NROW):
        for kk in range(8):
            off = j * 128 + kk * 16
            iav[pl.ds(off, 16)] = imgbase + (pav[pl.ds(off, 16)] & _OFF_MASK)
            ibv[pl.ds(off, 16)] = imgbase + (pbv[pl.ds(off, 16)] & _OFF_MASK)

    ga = pltpu.async_copy(img.at[iav], vav, sem)
    gb = pltpu.async_copy(img.at[ibv], vbv, sem)
    ga.wait()
    gb.wait()

    lane = lax.iota(jnp.int32, 16)

    def comp_body(j, acc):
        for kk in range(8):
            off = kk * 16
            a = vav[pl.ds(j * 128 + kk * 16, 16)]
            b = vbv[pl.ds(j * 128 + kk * 16, 16)]
            o = lax.shift_right_logical(pav[pl.ds(j * 128 + kk * 16, 16)], 18)
            diff = _vlog(a / b)
            r = (o - 1).astype(jnp.float32)
            t = jnp.float32(_MARGIN) - r * diff
            u = jnp.exp(-jnp.abs(t))
            sp = jnp.maximum(t, 0.0) + _vlog1p(u)
            eq = jnp.maximum(diff * diff - jnp.float32(_MARGIN), 0.0)
            per = jnp.where(o == 1, eq, sp)
            per = jnp.where(j * 128 + off + lane < limit, per, 0.0)
            acc = acc + per
        return acc

    acc = lax.fori_loop(0, _NROW, comp_body, jnp.zeros((16,), jnp.float32))

    resv[...] = acc * jnp.float32(_INV_P)
    pltpu.sync_copy(resv, out.at[wid])


@functools.partial(
    pl.kernel,
    mesh=plsc.VectorSubcoreMesh(core_axis_name="c", subcore_axis_name="s"),
    out_type=jax.ShapeDtypeStruct((_NW, 16), jnp.float32),
    scratch_types=[
        pltpu.VMEM((_CHUNK,), jnp.int32),
        pltpu.VMEM((_CHUNK,), jnp.int32),
        pltpu.VMEM((_CHUNK,), jnp.int32),
        pltpu.VMEM((_CHUNK,), jnp.int32),
        pltpu.VMEM((_CHUNK,), jnp.float32),
        pltpu.VMEM((_CHUNK,), jnp.float32),
        pltpu.VMEM((16,), jnp.float32),
        pltpu.SemaphoreType.DMA,
        pltpu.SemaphoreType.DMA((_NROW,)),
    ],
)
def _sc_loss(img, pa, pb, out, *scratch):
    _tec_body(img, pa, pb, out, *scratch)


def _pad_flat(a):
    return jnp.pad(a, ((0, 0), (0, _PPAD - _P))).reshape(-1)


def _tile_off(y, x):
    return (((y >> 3) * (_W // 128) + (x >> 7)) << 10) + ((y & 7) << 7) + (x & 127)


def kernel(input, y_A, x_A, y_B, x_B, ordinal):
    img = (input.reshape(_B, _H // 8, 8, _W // 128, 128)
           .transpose(0, 1, 3, 2, 4).reshape(-1))
    pa = _pad_flat(_tile_off(y_A, x_A) | (ordinal << 18))
    pb = _pad_flat(_tile_off(y_B, x_B))
    partials = _sc_loss(img, pa, pb)
    return jnp.sum(partials)


# speedup vs baseline: 1.2476x; 1.0150x over previous
import functools

import jax
import jax.numpy as jnp
from jax import lax
from jax.experimental import pallas as pl
from jax.experimental.pallas import tpu as pltpu
from jax.experimental.pallas import tpu_sc as plsc

_B = 8
_H = 384
_W = 384
_P = 5000
_PPAD = 5120
_NPAIR = _B * _PPAD
_CHUNK = 1280
_NROW = _CHUNK // 128
_NW = 32
_VALID_TAIL = _P - 3 * _CHUNK
_OFF_MASK = (1 << 18) - 1

_MARGIN = 0.25
_LN2 = 0.6931471805599453
_SQRT2 = 1.4142135623730951
_INV_P = 1.0 / _P


def _vlog(x):
    bits = lax.bitcast_convert_type(x, jnp.int32)
    e = lax.shift_right_arithmetic(bits, 23) - 127
    m_bits = lax.bitwise_or(lax.bitwise_and(bits, 0x007FFFFF), 0x3F800000)
    m = lax.bitcast_convert_type(m_bits, jnp.float32)
    big = m >= jnp.float32(_SQRT2)
    m = jnp.where(big, m * jnp.float32(0.5), m)
    e = e + jnp.where(big, 1, 0)
    ef = e.astype(jnp.float32)
    t = (m - 1.0) / (m + 1.0)
    t2 = t * t
    p = jnp.float32(1.0 / 9.0)
    p = p * t2 + jnp.float32(1.0 / 7.0)
    p = p * t2 + jnp.float32(1.0 / 5.0)
    p = p * t2 + jnp.float32(1.0 / 3.0)
    p = p * t2 + jnp.float32(1.0)
    return ef * jnp.float32(_LN2) + (t + t) * p


def _vlog1p(u):
    t = u / (u + 2.0)
    t2 = t * t
    p = jnp.float32(1.0 / 11.0)
    p = p * t2 + jnp.float32(1.0 / 9.0)
    p = p * t2 + jnp.float32(1.0 / 7.0)
    p = p * t2 + jnp.float32(1.0 / 5.0)
    p = p * t2 + jnp.float32(1.0 / 3.0)
    p = p * t2 + jnp.float32(1.0)
    return (t + t) * p


def _tec_body(img, pa, pb, out, pav, pbv, iav, ibv, vav, vbv, resv, sem, gsem):
    cid = lax.axis_index("c")
    sid = lax.axis_index("s")
    wid = sid * 2 + cid
    base = wid * _CHUNK
    imgbase = (wid // 4) * (_H * _W)
    limit = jnp.where(wid % 4 == 3, _VALID_TAIL, _CHUNK)

    c4 = wid % 4
    cps = []
    for u in range(_NROW):
        src = pl.multiple_of(((c4 * _NROW + u) << 10) + ((wid // 4) << 7), 8)
        cps.append(pltpu.async_copy(pa.at[pl.ds(src, 128)],
                                    pav.at[pl.ds(u * 128, 128)], sem))
        cps.append(pltpu.async_copy(pb.at[pl.ds(src, 128)],
                                    pbv.at[pl.ds(u * 128, 128)], sem))
    for cp in cps:
        cp.wait()

    for j in range(_NROW):
        for kk in range(8):
            off = j * 128 + kk * 16
            iav[pl.ds(off, 16)] = imgbase + (pav[pl.ds(off, 16)] & _OFF_MASK)
            ibv[pl.ds(off, 16)] = imgbase + (pbv[pl.ds(off, 16)] & _OFF_MASK)

    ga = pltpu.async_copy(img.at[iav], vav, sem)
    gb = pltpu.async_copy(img.at[ibv], vbv, sem)
    ga.wait()
    gb.wait()

    lane = lax.iota(jnp.int32, 16)

    def comp_body(j, acc):
        for kk in range(8):
            off = kk * 16
            a = vav[pl.ds(j * 128 + kk * 16, 16)]
            b = vbv[pl.ds(j * 128 + kk * 16, 16)]
            o = lax.shift_right_logical(pav[pl.ds(j * 128 + kk * 16, 16)], 18)
            diff = _vlog(a / b)
            r = (o - 1).astype(jnp.float32)
            t = jnp.float32(_MARGIN) - r * diff
            u = jnp.exp(-jnp.abs(t))
            sp = jnp.maximum(t, 0.0) + _vlog1p(u)
            eq = jnp.maximum(diff * diff - jnp.float32(_MARGIN), 0.0)
            per = jnp.where(o == 1, eq, sp)
            per = jnp.where(j * 128 + off + lane < limit, per, 0.0)
            acc = acc + per
        return acc

    acc = lax.fori_loop(0, _NROW, comp_body, jnp.zeros((16,), jnp.float32))

    resv[...] = acc * jnp.float32(_INV_P)
    pltpu.sync_copy(resv, out.at[wid])


@functools.partial(
    pl.kernel,
    mesh=plsc.VectorSubcoreMesh(core_axis_name="c", subcore_axis_name="s"),
    out_type=jax.ShapeDtypeStruct((_NW, 16), jnp.float32),
    scratch_types=[
        pltpu.VMEM((_CHUNK,), jnp.int32),
        pltpu.VMEM((_CHUNK,), jnp.int32),
        pltpu.VMEM((_CHUNK,), jnp.int32),
        pltpu.VMEM((_CHUNK,), jnp.int32),
        pltpu.VMEM((_CHUNK,), jnp.float32),
        pltpu.VMEM((_CHUNK,), jnp.float32),
        pltpu.VMEM((16,), jnp.float32),
        pltpu.SemaphoreType.DMA,
        pltpu.SemaphoreType.DMA((_NROW,)),
    ],
)
def _sc_loss(img, pa, pb, out, *scratch):
    _tec_body(img, pa, pb, out, *scratch)


def _pad_flat(a):
    return (jnp.pad(a, ((0, 0), (0, _PPAD - _P)))
            .reshape(_B, _PPAD // 128, 128).transpose(1, 0, 2).reshape(-1))


def _tile_off(y, x):
    return (((y >> 3) * (_W // 128) + (x >> 7)) << 10) + ((y & 7) << 7) + (x & 127)


def kernel(input, y_A, x_A, y_B, x_B, ordinal):
    img = (input.reshape(_B, _H // 8, 8, _W // 128, 128)
           .transpose(0, 1, 3, 2, 4).reshape(-1))
    pa = _pad_flat(_tile_off(y_A, x_A) | (ordinal << 18))
    pb = _pad_flat(_tile_off(y_B, x_B))
    partials = _sc_loss(img, pa, pb)
    return jnp.sum(partials)
